# baseline (speedup 1.0000x reference)
import functools

import jax
import jax.numpy as jnp
from jax import lax
from jax.experimental import pallas as pl
from jax.experimental.pallas import tpu as pltpu
from jax.experimental.pallas import tpu_sc as plsc

T = 2048
D = 1024
DFF = 2048
E = 8
K = 2
C = 640
S = E * C
S_PAD = 5376
N_TRASH = S_PAD - S
XROWS = T + 8
NB = S_PAD // 128
NJ = 4
FJ = DFF // NJ

NC = 2
NS = 16
HALF = S_PAD // NC
PER_TILE = HALF // NS
INIT_PER_TILE = S_PAD // NS


def _router_body(x_ref, wg_ref, out_ref):
    x = x_ref[...]
    logits = jnp.dot(x, wg_ref[...], preferred_element_type=jnp.float32)
    col = lax.broadcasted_iota(jnp.int32, (T, 128), 1)
    valid = col < E
    lg = jnp.where(valid, logits, -3e38)
    m = jnp.max(lg, axis=1, keepdims=True)
    p = jnp.where(valid, jnp.exp(lg - m), 0.0)
    probs = p / jnp.sum(p, axis=1, keepdims=True)
    p0 = jnp.max(probs, axis=1, keepdims=True)
    i0 = jnp.min(jnp.where(probs == p0, col, 128), axis=1, keepdims=True)
    mask0 = col == i0
    probs2 = jnp.where(mask0, -1.0, probs)
    p1 = jnp.max(probs2, axis=1, keepdims=True)
    i1 = jnp.min(jnp.where(probs2 == p1, col, 128), axis=1, keepdims=True)
    mask1 = col == i1
    sm = p0 + p1 + 1e-9
    g0 = p0 / sm
    g1 = p1 / sm
    cnt = mask0.astype(jnp.float32) + mask1.astype(jnp.float32)
    ri = lax.broadcasted_iota(jnp.int32, (T, T), 0)
    cj = lax.broadcasted_iota(jnp.int32, (T, T), 1)
    tri = (cj < ri).astype(jnp.bfloat16)
    pref = jnp.dot(tri, cnt.astype(jnp.bfloat16),
                   preferred_element_type=jnp.float32)
    pos0 = jnp.sum(jnp.where(mask0, pref, 0.0), axis=1, keepdims=True)
    pos1 = jnp.sum(jnp.where(mask1, pref, 0.0), axis=1, keepdims=True)
    keep0 = pos0 < C
    keep1 = pos1 < C
    t2 = lax.broadcasted_iota(jnp.int32, (T, 1), 0) * 2
    trash0 = (S + (t2 % N_TRASH)).astype(jnp.float32)
    trash1 = (S + ((t2 + 1) % N_TRASH)).astype(jnp.float32)
    i0f = i0.astype(jnp.float32)
    i1f = i1.astype(jnp.float32)
    d0 = jnp.where(keep0, i0f * C + pos0, trash0)
    d1 = jnp.where(keep1, i1f * C + pos1, trash1)
    gv0 = jnp.where(keep0, g0, 0.0)
    gv1 = jnp.where(keep1, g1, 0.0)
    packed = jnp.where(col == 0, d0,
             jnp.where(col == 1, d1,
             jnp.where(col == 2, gv0,
             jnp.where(col == 3, gv1, 0.0))))
    out_ref[...] = packed


def _run_router(x, wg_pad):
    return pl.pallas_call(
        _router_body,
        out_shape=jax.ShapeDtypeStruct((T, 128), jnp.float32),
    )(x, wg_pad)


def _dispatch_body(dest_hbm, tok_hbm, gate_hbm, tfsinit_hbm, gateinit_hbm,
                   xpad_hbm, buf_hbm, gateout_hbm,
                   tfs_sp, gate_sp, idx_v, tok_v, gval_v, tfs_v, rows_v,
                   binit_i, binit_f, gout_v, sem):
    cid = lax.axis_index("c")
    sid = lax.axis_index("s")
    o = sid * INIT_PER_TILE
    pltpu.sync_copy(tfsinit_hbm.at[pl.ds(o, INIT_PER_TILE)], binit_i)
    pltpu.sync_copy(binit_i, tfs_sp.at[pl.ds(o, INIT_PER_TILE)])
    pltpu.sync_copy(gateinit_hbm.at[pl.ds(o, INIT_PER_TILE)], binit_f)
    pltpu.sync_copy(binit_f, gate_sp.at[pl.ds(o, INIT_PER_TILE)])
    plsc.subcore_barrier()
    r0 = sid * 2
    pltpu.sync_copy(dest_hbm.at[pl.ds(r0, 2)], idx_v)
    pltpu.sync_copy(tok_hbm.at[pl.ds(r0, 2)], tok_v)
    pltpu.sync_copy(gate_hbm.at[pl.ds(r0, 2)], gval_v)
    for j in range(2):
        pltpu.sync_copy(tok_v.at[j], tfs_sp.at[idx_v.at[j]])
        pltpu.sync_copy(gval_v.at[j], gate_sp.at[idx_v.at[j]])
    plsc.subcore_barrier()
    g0 = cid * HALF + sid * PER_TILE
    pltpu.sync_copy(tfs_sp.at[pl.ds(g0, PER_TILE)], tfs_v)
    for off, sz in ((0, 88), (88, 80)):
        pltpu.async_copy(xpad_hbm.at[tfs_v.at[pl.ds(off, sz)]],
                         rows_v.at[pl.ds(0, sz)], sem).wait()
        pltpu.sync_copy(rows_v.at[pl.ds(0, sz)],
                        buf_hbm.at[pl.ds(g0 + off, sz)])
    pltpu.sync_copy(gate_sp.at[pl.ds(g0, PER_TILE)], gout_v)
    pltpu.sync_copy(gout_v, gateout_hbm.at[pl.ds(g0, PER_TILE)])


def _run_dispatch(dest32, tok32, gate32, tfs_init, gate_init, x_pad):
    mesh = plsc.VectorSubcoreMesh(core_axis_name="c", subcore_axis_name="s")
    f = pl.kernel(
        _dispatch_body,
        out_type=[
            jax.ShapeDtypeStruct((S_PAD, D), jnp.float32),
            jax.ShapeDtypeStruct((S_PAD,), jnp.float32),
        ],
        mesh=mesh,
        scratch_types=[
            pltpu.VMEM_SHARED((S_PAD,), jnp.int32),
            pltpu.VMEM_SHARED((S_PAD,), jnp.float32),
            pltpu.VMEM((2, 128), jnp.int32),
            pltpu.VMEM((2, 128), jnp.int32),
            pltpu.VMEM((2, 128), jnp.float32),
            pltpu.VMEM((PER_TILE,), jnp.int32),
            pltpu.VMEM((88, D), jnp.float32),
            pltpu.VMEM((INIT_PER_TILE,), jnp.int32),
            pltpu.VMEM((INIT_PER_TILE,), jnp.float32),
            pltpu.VMEM((PER_TILE,), jnp.float32),
            pltpu.SemaphoreType.DMA,
        ],
    )
    return f(dest32, tok32, gate32, tfs_init, gate_init, x_pad)


def _ffn_body(buf_ref, w1_ref, b1_ref, w2_ref, b2_ref, gate_ref, out_ref):
    xb = buf_ref[...]
    h = (jnp.dot(xb, w1_ref[0], preferred_element_type=jnp.float32)
         + b1_ref[0, 0, :])
    h = jax.nn.gelu(h)
    out_ref[...] = (jnp.dot(h.astype(jnp.bfloat16), w2_ref[0],
                            preferred_element_type=jnp.float32)
                    + b2_ref[0, 0, :]) * gate_ref[:, 0:1]


def _run_ffn(buf, w1, b1r, w2, b2r, gate_b):
    emap = lambda i: jnp.minimum(i // 5, E - 1)
    return pl.pallas_call(
        _ffn_body,
        grid=(NB,),
        in_specs=[
            pl.BlockSpec((128, D), lambda i: (i, 0)),
            pl.BlockSpec((1, D, DFF), lambda i: (emap(i), 0, 0)),
            pl.BlockSpec((1, 1, DFF), lambda i: (emap(i), 0, 0)),
            pl.BlockSpec((1, DFF, D), lambda i: (emap(i), 0, 0)),
            pl.BlockSpec((1, 1, D), lambda i: (emap(i), 0, 0)),
            pl.BlockSpec((128, 128), lambda i: (i, 0)),
        ],
        out_specs=pl.BlockSpec((128, D), lambda i: (i, 0)),
        out_shape=jax.ShapeDtypeStruct((S_PAD, D), jnp.float32),
        compiler_params=pltpu.CompilerParams(
            dimension_semantics=("arbitrary",)),
    )(buf, w1, b1r, w2, b2r, gate_b)


def _combine_body(cidx_hbm, scaled_hbm, out_hbm, idx_v, rows0, rows1, sem):
    cid = lax.axis_index("c")
    sid = lax.axis_index("s")
    wid = sid * NC + cid
    pltpu.sync_copy(cidx_hbm.at[pl.ds(wid, 1)], idx_v)
    for c in range(2):
        pltpu.async_copy(scaled_hbm.at[idx_v.at[0, pl.ds(c * 32, 32)]],
                         rows0, sem).wait()
        pltpu.async_copy(scaled_hbm.at[idx_v.at[0, pl.ds(64 + c * 32, 32)]],
                         rows1, sem).wait()

        def add_row(r, _):
            for v in range(D // 16):
                sl = pl.ds(v * 16, 16)
                rows0[r, sl] = rows0[r, sl] + rows1[r, sl]
            return 0

        lax.fori_loop(0, 32, add_row, 0)
        pltpu.sync_copy(rows0, out_hbm.at[pl.ds(wid * 64 + c * 32, 32)])


def _run_combine(cidx32, scaled):
    mesh = plsc.VectorSubcoreMesh(core_axis_name="c", subcore_axis_name="s")
    f = pl.kernel(
        _combine_body,
        out_type=jax.ShapeDtypeStruct((T, D), jnp.float32),
        mesh=mesh,
        scratch_types=[
            pltpu.VMEM((1, 128), jnp.int32),
            pltpu.VMEM((32, D), jnp.float32),
            pltpu.VMEM((32, D), jnp.float32),
            pltpu.SemaphoreType.DMA,
        ],
    )
    return f(cidx32, scaled)


@jax.jit
def kernel(x, Wg, W1, b1, W2, b2):
    wg_pad = jnp.zeros((D, 128), jnp.float32).at[:, :E].set(Wg)
    x_pad = jnp.concatenate(
        [x, jnp.zeros((XROWS - T, D), jnp.float32)], axis=0)

    packed = _run_router(x, wg_pad)
    d0 = packed[:, 0]
    d1 = packed[:, 1]
    gv0 = packed[:, 2]
    gv1 = packed[:, 3]
    dest32 = jnp.concatenate(
        [d0.reshape(32, 64), d1.reshape(32, 64)], axis=1).astype(jnp.int32)
    tok = jnp.arange(T, dtype=jnp.int32).reshape(32, 64)
    tok32 = jnp.concatenate([tok, tok], axis=1)
    gate32 = jnp.concatenate([gv0.reshape(32, 64), gv1.reshape(32, 64)],
                             axis=1)

    tfs_init = jnp.full((S_PAD,), T, jnp.int32)
    gate_init = jnp.zeros((S_PAD,), jnp.float32)
    buf, gate_slot = _run_dispatch(dest32, tok32, gate32, tfs_init,
                                   gate_init, x_pad)

    b1r = b1.reshape(E, 1, DFF)
    b2r = b2.reshape(E, 1, D)
    gate_b = jnp.broadcast_to(gate_slot.reshape(S_PAD, 1), (S_PAD, 128))
    scaled = _run_ffn(buf.astype(jnp.bfloat16), W1.astype(jnp.bfloat16),
                      b1r, W2.astype(jnp.bfloat16), b2r, gate_b)

    return _run_combine(dest32, scaled)


# speedup vs baseline: 1.1610x; 1.1610x over previous
import functools

import jax
import jax.numpy as jnp
from jax import lax
from jax.experimental import pallas as pl
from jax.experimental.pallas import tpu as pltpu
from jax.experimental.pallas import tpu_sc as plsc

T = 2048
D = 1024
DFF = 2048
E = 8
K = 2
C = 640
S = E * C
S_PAD = 5376
N_TRASH = S_PAD - S
XROWS = T + 8
NB = S_PAD // 128
NJ = 4
FJ = DFF // NJ

NC = 2
NS = 16
HALF = S_PAD // NC
PER_TILE = HALF // NS
INIT_PER_TILE = S_PAD // NS


def _router_body(x_ref, wg_ref, out_ref):
    x = x_ref[...]
    logits = jnp.dot(x, wg_ref[...], preferred_element_type=jnp.float32)
    col = lax.broadcasted_iota(jnp.int32, (T, 128), 1)
    valid = col < E
    lg = jnp.where(valid, logits, -3e38)
    m = jnp.max(lg, axis=1, keepdims=True)
    p = jnp.where(valid, jnp.exp(lg - m), 0.0)
    probs = p / jnp.sum(p, axis=1, keepdims=True)
    p0 = jnp.max(probs, axis=1, keepdims=True)
    i0 = jnp.min(jnp.where(probs == p0, col, 128), axis=1, keepdims=True)
    mask0 = col == i0
    probs2 = jnp.where(mask0, -1.0, probs)
    p1 = jnp.max(probs2, axis=1, keepdims=True)
    i1 = jnp.min(jnp.where(probs2 == p1, col, 128), axis=1, keepdims=True)
    mask1 = col == i1
    sm = p0 + p1 + 1e-9
    g0 = p0 / sm
    g1 = p1 / sm
    cnt = mask0.astype(jnp.float32) + mask1.astype(jnp.float32)
    ri = lax.broadcasted_iota(jnp.int32, (T, T), 0)
    cj = lax.broadcasted_iota(jnp.int32, (T, T), 1)
    tri = (cj < ri).astype(jnp.bfloat16)
    pref = jnp.dot(tri, cnt.astype(jnp.bfloat16),
                   preferred_element_type=jnp.float32)
    pos0 = jnp.sum(jnp.where(mask0, pref, 0.0), axis=1, keepdims=True)
    pos1 = jnp.sum(jnp.where(mask1, pref, 0.0), axis=1, keepdims=True)
    keep0 = pos0 < C
    keep1 = pos1 < C
    t2 = lax.broadcasted_iota(jnp.int32, (T, 1), 0) * 2
    trash0 = (S + (t2 % N_TRASH)).astype(jnp.float32)
    trash1 = (S + ((t2 + 1) % N_TRASH)).astype(jnp.float32)
    i0f = i0.astype(jnp.float32)
    i1f = i1.astype(jnp.float32)
    d0 = jnp.where(keep0, i0f * C + pos0, trash0)
    d1 = jnp.where(keep1, i1f * C + pos1, trash1)
    gv0 = jnp.where(keep0, g0, 0.0)
    gv1 = jnp.where(keep1, g1, 0.0)
    packed = jnp.where(col == 0, d0,
             jnp.where(col == 1, d1,
             jnp.where(col == 2, gv0,
             jnp.where(col == 3, gv1, 0.0))))
    out_ref[...] = packed


def _run_router(x, wg_pad):
    return pl.pallas_call(
        _router_body,
        out_shape=jax.ShapeDtypeStruct((T, 128), jnp.float32),
    )(x, wg_pad)


def _dispatch_body(dest_hbm, tok_hbm, gate_hbm, tfsinit_hbm, gateinit_hbm,
                   xpad_hbm, buf_hbm, gateout_hbm,
                   tfs_sp, gate_sp, idx_v, tok_v, gval_v, tfs_v, rows_v,
                   binit_i, binit_f, gout_v, sem):
    cid = lax.axis_index("c")
    sid = lax.axis_index("s")
    o = sid * INIT_PER_TILE
    pltpu.sync_copy(tfsinit_hbm.at[pl.ds(o, INIT_PER_TILE)], binit_i)
    pltpu.sync_copy(binit_i, tfs_sp.at[pl.ds(o, INIT_PER_TILE)])
    pltpu.sync_copy(gateinit_hbm.at[pl.ds(o, INIT_PER_TILE)], binit_f)
    pltpu.sync_copy(binit_f, gate_sp.at[pl.ds(o, INIT_PER_TILE)])
    plsc.subcore_barrier()
    r0 = sid * 2
    pltpu.sync_copy(dest_hbm.at[pl.ds(r0, 2)], idx_v)
    pltpu.sync_copy(tok_hbm.at[pl.ds(r0, 2)], tok_v)
    pltpu.sync_copy(gate_hbm.at[pl.ds(r0, 2)], gval_v)
    for j in range(2):
        pltpu.sync_copy(tok_v.at[j], tfs_sp.at[idx_v.at[j]])
        pltpu.sync_copy(gval_v.at[j], gate_sp.at[idx_v.at[j]])
    plsc.subcore_barrier()
    g0 = cid * HALF + sid * PER_TILE
    pltpu.sync_copy(tfs_sp.at[pl.ds(g0, PER_TILE)], tfs_v)
    for off, sz in ((0, 88), (88, 80)):
        pltpu.async_copy(xpad_hbm.at[tfs_v.at[pl.ds(off, sz)]],
                         rows_v.at[pl.ds(0, sz)], sem).wait()
        pltpu.sync_copy(rows_v.at[pl.ds(0, sz)],
                        buf_hbm.at[pl.ds(g0 + off, sz)])
    pltpu.sync_copy(gate_sp.at[pl.ds(g0, PER_TILE)], gout_v)
    pltpu.sync_copy(gout_v, gateout_hbm.at[pl.ds(g0, PER_TILE)])


def _run_dispatch(dest32, tok32, gate32, tfs_init, gate_init, x_pad):
    mesh = plsc.VectorSubcoreMesh(core_axis_name="c", subcore_axis_name="s")
    f = pl.kernel(
        _dispatch_body,
        out_type=[
            jax.ShapeDtypeStruct((S_PAD, D), jnp.float32),
            jax.ShapeDtypeStruct((S_PAD,), jnp.float32),
        ],
        mesh=mesh,
        scratch_types=[
            pltpu.VMEM_SHARED((S_PAD,), jnp.int32),
            pltpu.VMEM_SHARED((S_PAD,), jnp.float32),
            pltpu.VMEM((2, 128), jnp.int32),
            pltpu.VMEM((2, 128), jnp.int32),
            pltpu.VMEM((2, 128), jnp.float32),
            pltpu.VMEM((PER_TILE,), jnp.int32),
            pltpu.VMEM((88, D), jnp.float32),
            pltpu.VMEM((INIT_PER_TILE,), jnp.int32),
            pltpu.VMEM((INIT_PER_TILE,), jnp.float32),
            pltpu.VMEM((PER_TILE,), jnp.float32),
            pltpu.SemaphoreType.DMA,
        ],
    )
    return f(dest32, tok32, gate32, tfs_init, gate_init, x_pad)


def _ffn_body(buf_ref, w1_ref, b1_ref, w2_ref, b2_ref, gate_ref, out_ref):
    xb = buf_ref[...]
    h = (jnp.dot(xb, w1_ref[0], preferred_element_type=jnp.float32)
         + b1_ref[0, 0, :])
    h = jax.nn.gelu(h)
    out_ref[...] = (jnp.dot(h, w2_ref[0], preferred_element_type=jnp.float32)
                    + b2_ref[0, 0, :]) * gate_ref[:, 0:1]


def _run_ffn(buf, w1, b1r, w2, b2r, gate_b):
    emap = lambda i: jnp.minimum(i // 5, E - 1)
    return pl.pallas_call(
        _ffn_body,
        grid=(NB,),
        in_specs=[
            pl.BlockSpec((128, D), lambda i: (i, 0)),
            pl.BlockSpec((1, D, DFF), lambda i: (emap(i), 0, 0)),
            pl.BlockSpec((1, 1, DFF), lambda i: (emap(i), 0, 0)),
            pl.BlockSpec((1, DFF, D), lambda i: (emap(i), 0, 0)),
            pl.BlockSpec((1, 1, D), lambda i: (emap(i), 0, 0)),
            pl.BlockSpec((128, 128), lambda i: (i, 0)),
        ],
        out_specs=pl.BlockSpec((128, D), lambda i: (i, 0)),
        out_shape=jax.ShapeDtypeStruct((S_PAD, D), jnp.float32),
        compiler_params=pltpu.CompilerParams(
            dimension_semantics=("arbitrary",)),
    )(buf, w1, b1r, w2, b2r, gate_b)


def _combine_body(cidx_hbm, scaled_hbm, out_hbm, idx_v, rows0, rows1, sem):
    cid = lax.axis_index("c")
    sid = lax.axis_index("s")
    wid = sid * NC + cid
    pltpu.sync_copy(cidx_hbm.at[pl.ds(wid, 1)], idx_v)
    for c in range(2):
        pltpu.async_copy(scaled_hbm.at[idx_v.at[0, pl.ds(c * 32, 32)]],
                         rows0, sem).wait()
        pltpu.async_copy(scaled_hbm.at[idx_v.at[0, pl.ds(64 + c * 32, 32)]],
                         rows1, sem).wait()

        def add_row(r, _):
            for v in range(D // 16):
                sl = pl.ds(v * 16, 16)
                rows0[r, sl] = rows0[r, sl] + rows1[r, sl]
            return 0

        lax.fori_loop(0, 32, add_row, 0)
        pltpu.sync_copy(rows0, out_hbm.at[pl.ds(wid * 64 + c * 32, 32)])


def _run_combine(cidx32, scaled):
    mesh = plsc.VectorSubcoreMesh(core_axis_name="c", subcore_axis_name="s")
    f = pl.kernel(
        _combine_body,
        out_type=jax.ShapeDtypeStruct((T, D), jnp.float32),
        mesh=mesh,
        scratch_types=[
            pltpu.VMEM((1, 128), jnp.int32),
            pltpu.VMEM((32, D), jnp.float32),
            pltpu.VMEM((32, D), jnp.float32),
            pltpu.SemaphoreType.DMA,
        ],
    )
    return f(cidx32, scaled)


@jax.jit
def kernel(x, Wg, W1, b1, W2, b2):
    wg_pad = jnp.zeros((D, 128), jnp.float32).at[:, :E].set(Wg)
    x_pad = jnp.concatenate(
        [x, jnp.zeros((XROWS - T, D), jnp.float32)], axis=0)

    packed = _run_router(x, wg_pad)
    d0 = packed[:, 0]
    d1 = packed[:, 1]
    gv0 = packed[:, 2]
    gv1 = packed[:, 3]
    dest32 = jnp.concatenate(
        [d0.reshape(32, 64), d1.reshape(32, 64)], axis=1).astype(jnp.int32)
    tok = jnp.arange(T, dtype=jnp.int32).reshape(32, 64)
    tok32 = jnp.concatenate([tok, tok], axis=1)
    gate32 = jnp.concatenate([gv0.reshape(32, 64), gv1.reshape(32, 64)],
                             axis=1)

    tfs_init = jnp.full((S_PAD,), T, jnp.int32)
    gate_init = jnp.zeros((S_PAD,), jnp.float32)
    buf, gate_slot = _run_dispatch(dest32, tok32, gate32, tfs_init,
                                   gate_init, x_pad)

    b1r = b1.reshape(E, 1, DFF)
    b2r = b2.reshape(E, 1, D)
    gate_b = jnp.broadcast_to(gate_slot.reshape(S_PAD, 1), (S_PAD, 128))
    scaled = _run_ffn(buf, W1, b1r, W2, b2r, gate_b)

    return _run_combine(dest32, scaled)


# speedup vs baseline: 1.1929x; 1.0275x over previous
import jax
import jax.numpy as jnp
from jax import lax
from jax.experimental import pallas as pl
from jax.experimental.pallas import tpu as pltpu
from jax.experimental.pallas import tpu_sc as plsc

T = 2048
D = 1024
DFF = 2048
E = 8
C = 640
S = E * C
S_PAD = 5376
N_TRASH = S_PAD - S
NB = S_PAD // 128

NC = 2
NS = 16
HALF = S_PAD // NC
PER_TILE = HALF // NS
INIT_PER_TILE = S_PAD // NS


def _router_body(x_ref, wg_ref, dest_ref, gate_ref):
    x = x_ref[...]
    logits = jnp.dot(x, wg_ref[...], preferred_element_type=jnp.float32)
    col = lax.broadcasted_iota(jnp.int32, (T, E), 1)
    m = jnp.max(logits, axis=1, keepdims=True)
    p = jnp.exp(logits - m)
    probs = p / jnp.sum(p, axis=1, keepdims=True)
    p0 = jnp.max(probs, axis=1, keepdims=True)
    i0 = jnp.min(jnp.where(probs == p0, col, E), axis=1, keepdims=True)
    mask0 = col == i0
    probs2 = jnp.where(mask0, -1.0, probs)
    p1 = jnp.max(probs2, axis=1, keepdims=True)
    i1 = jnp.min(jnp.where(probs2 == p1, col, E), axis=1, keepdims=True)
    mask1 = col == i1
    sm = p0 + p1 + 1e-9
    g0 = p0 / sm
    g1 = p1 / sm
    cnt = mask0.astype(jnp.float32) + mask1.astype(jnp.float32)
    ri = lax.broadcasted_iota(jnp.int32, (T, T), 0)
    cj = lax.broadcasted_iota(jnp.int32, (T, T), 1)
    tri = (cj < ri).astype(jnp.bfloat16)
    pref = jnp.dot(tri, cnt.astype(jnp.bfloat16),
                   preferred_element_type=jnp.float32)
    pos0 = jnp.sum(jnp.where(mask0, pref, 0.0), axis=1,
                   keepdims=True).astype(jnp.int32)
    pos1 = jnp.sum(jnp.where(mask1, pref, 0.0), axis=1,
                   keepdims=True).astype(jnp.int32)
    keep0 = pos0 < C
    keep1 = pos1 < C
    t2 = lax.broadcasted_iota(jnp.int32, (T, 1), 0) * 2
    d0 = jnp.where(keep0, i0 * C + pos0, S + (t2 % N_TRASH))
    d1 = jnp.where(keep1, i1 * C + pos1, S + ((t2 + 1) % N_TRASH))
    gv0 = jnp.where(keep0, g0, 0.0)
    gv1 = jnp.where(keep1, g1, 0.0)
    dest_ref[...] = jnp.concatenate([d0, d1], axis=1)
    gate_ref[...] = jnp.concatenate([gv0, gv1], axis=1)


def _run_router(x, Wg):
    return pl.pallas_call(
        _router_body,
        out_shape=[jax.ShapeDtypeStruct((T, 2), jnp.int32),
                   jax.ShapeDtypeStruct((T, 2), jnp.float32)],
    )(x, Wg)


def _dispatch_body(dest_hbm, tok_hbm, gate_hbm, x_hbm, buf_hbm, gateout_hbm,
                   tfs_sp, gate_sp, idx_v, tok_v, gval_v, tfs_v,
                   rows_a, rows_b, binit_i, binit_f, gout_v,
                   sem_in, sem_sp, sem_ga, sem_gb, sem_wa, sem_wb):
    cid = lax.axis_index("c")
    sid = lax.axis_index("s")
    r0 = sid * 2
    c1 = pltpu.async_copy(dest_hbm.at[pl.ds(r0, 2)], idx_v, sem_in)
    c2 = pltpu.async_copy(tok_hbm.at[pl.ds(r0, 2)], tok_v, sem_in)
    c3 = pltpu.async_copy(gate_hbm.at[pl.ds(r0, 2)], gval_v, sem_in)
    zi = jnp.zeros((16,), jnp.int32)
    zf = jnp.zeros((16,), jnp.float32)
    for k in range(INIT_PER_TILE // 16):
        binit_i[pl.ds(k * 16, 16)] = zi
        binit_f[pl.ds(k * 16, 16)] = zf
    o = sid * INIT_PER_TILE
    i1 = pltpu.async_copy(binit_i, tfs_sp.at[pl.ds(o, INIT_PER_TILE)], sem_sp)
    i2 = pltpu.async_copy(binit_f, gate_sp.at[pl.ds(o, INIT_PER_TILE)], sem_sp)
    c1.wait()
    c2.wait()
    c3.wait()
    i1.wait()
    i2.wait()
    plsc.subcore_barrier()
    scs = []
    for j in range(2):
        scs.append(pltpu.async_copy(tok_v.at[j], tfs_sp.at[idx_v.at[j]],
                                    sem_sp))
        scs.append(pltpu.async_copy(gval_v.at[j], gate_sp.at[idx_v.at[j]],
                                    sem_sp))
    for s in scs:
        s.wait()
    plsc.subcore_barrier()
    g0 = cid * HALF + sid * PER_TILE
    pltpu.sync_copy(tfs_sp.at[pl.ds(g0, PER_TILE)], tfs_v)
    cg = pltpu.async_copy(gate_sp.at[pl.ds(g0, PER_TILE)], gout_v, sem_in)
    offs = (0, 48, 96, 144)
    szs = (48, 48, 48, 24)
    bufs = (rows_a, rows_b)
    gsems = (sem_ga, sem_gb)
    wsems = (sem_wa, sem_wb)

    def start_gather(c):
        p = c % 2
        return pltpu.async_copy(
            x_hbm.at[tfs_v.at[pl.ds(offs[c], szs[c])]],
            bufs[p].at[pl.ds(0, szs[c])], gsems[p])

    def start_write(c):
        p = c % 2
        return pltpu.async_copy(
            bufs[p].at[pl.ds(0, szs[c])],
            buf_hbm.at[pl.ds(g0 + offs[c], szs[c])], wsems[p])

    g_a = start_gather(0)
    g_b = start_gather(1)
    g_a.wait()
    w_a = start_write(0)
    g_b.wait()
    w_b = start_write(1)
    w_a.wait()
    g_a = start_gather(2)
    w_b.wait()
    g_b = start_gather(3)
    g_a.wait()
    w_a = start_write(2)
    g_b.wait()
    w_b = start_write(3)
    w_a.wait()
    w_b.wait()
    cg.wait()
    pltpu.sync_copy(gout_v, gateout_hbm.at[pl.ds(g0, PER_TILE)])


def _run_dispatch(dest32, tok32, gate32, x):
    mesh = plsc.VectorSubcoreMesh(core_axis_name="c", subcore_axis_name="s")
    f = pl.kernel(
        _dispatch_body,
        out_type=[
            jax.ShapeDtypeStruct((S_PAD, D), jnp.float32),
            jax.ShapeDtypeStruct((S_PAD,), jnp.float32),
        ],
        mesh=mesh,
        scratch_types=[
            pltpu.VMEM_SHARED((S_PAD,), jnp.int32),
            pltpu.VMEM_SHARED((S_PAD,), jnp.float32),
            pltpu.VMEM((2, 128), jnp.int32),
            pltpu.VMEM((2, 128), jnp.int32),
            pltpu.VMEM((2, 128), jnp.float32),
            pltpu.VMEM((PER_TILE,), jnp.int32),
            pltpu.VMEM((48, D), jnp.float32),
            pltpu.VMEM((48, D), jnp.float32),
            pltpu.VMEM((INIT_PER_TILE,), jnp.int32),
            pltpu.VMEM((INIT_PER_TILE,), jnp.float32),
            pltpu.VMEM((PER_TILE,), jnp.float32),
            pltpu.SemaphoreType.DMA,
            pltpu.SemaphoreType.DMA,
            pltpu.SemaphoreType.DMA,
            pltpu.SemaphoreType.DMA,
            pltpu.SemaphoreType.DMA,
            pltpu.SemaphoreType.DMA,
        ],
    )
    return f(dest32, tok32, gate32, x)


def _ffn_body(buf_ref, w1_ref, b1_ref, w2_ref, b2_ref, gate_ref, out_ref):
    xb = buf_ref[...]
    h = (jnp.dot(xb, w1_ref[0], preferred_element_type=jnp.float32)
         + b1_ref[0, 0, :])
    h = jax.nn.gelu(h)
    out_ref[...] = (jnp.dot(h, w2_ref[0], preferred_element_type=jnp.float32)
                    + b2_ref[0, 0, :]) * gate_ref[...]


def _run_ffn(buf, w1, b1r, w2, b2r, gate_col):
    emap = lambda i: jnp.minimum(i // 5, E - 1)
    return pl.pallas_call(
        _ffn_body,
        grid=(NB,),
        in_specs=[
            pl.BlockSpec((128, D), lambda i: (i, 0)),
            pl.BlockSpec((1, D, DFF), lambda i: (emap(i), 0, 0)),
            pl.BlockSpec((1, 1, DFF), lambda i: (emap(i), 0, 0)),
            pl.BlockSpec((1, DFF, D), lambda i: (emap(i), 0, 0)),
            pl.BlockSpec((1, 1, D), lambda i: (emap(i), 0, 0)),
            pl.BlockSpec((128, 1), lambda i: (i, 0)),
        ],
        out_specs=pl.BlockSpec((128, D), lambda i: (i, 0)),
        out_shape=jax.ShapeDtypeStruct((S_PAD, D), jnp.float32),
        compiler_params=pltpu.CompilerParams(
            dimension_semantics=("arbitrary",)),
    )(buf, w1, b1r, w2, b2r, gate_col)


def _combine_body(cidx_hbm, scaled_hbm, out_hbm, idx_v,
                  rows_a, rows_b, oa, ob,
                  sem_ga, sem_gb, sem_wa, sem_wb):
    cid = lax.axis_index("c")
    sid = lax.axis_index("s")
    wid = sid * NC + cid
    pltpu.sync_copy(cidx_hbm.at[pl.ds(wid, 1)], idx_v)
    rows = (rows_a, rows_b)
    outs = (oa, ob)
    gsems = (sem_ga, sem_gb)
    wsems = (sem_wa, sem_wb)

    def start_gather(c):
        p = c % 2
        return pltpu.async_copy(
            scaled_hbm.at[idx_v.at[0, pl.ds(c * 32, 32)]], rows[p], gsems[p])

    def add_and_write(c):
        p = c % 2

        def add_row(r, _):
            for v in range(D // 16):
                sl = pl.ds(v * 16, 16)
                outs[p][r, sl] = rows[p][2 * r, sl] + rows[p][2 * r + 1, sl]
            return 0

        lax.fori_loop(0, 16, add_row, 0)
        return pltpu.async_copy(
            outs[p], out_hbm.at[pl.ds(wid * 64 + c * 16, 16)], wsems[p])

    pend_g = [None, None]
    pend_w = [None, None]
    pend_g[0] = start_gather(0)
    pend_g[1] = start_gather(1)
    for c in range(4):
        p = c % 2
        pend_g[p].wait()
        if pend_w[p] is not None:
            pend_w[p].wait()
        pend_w[p] = add_and_write(c)
        if c + 2 < 4:
            pend_g[p] = start_gather(c + 2)
    pend_w[0].wait()
    pend_w[1].wait()


def _run_combine(cidx32, scaled):
    mesh = plsc.VectorSubcoreMesh(core_axis_name="c", subcore_axis_name="s")
    f = pl.kernel(
        _combine_body,
        out_type=jax.ShapeDtypeStruct((T, D), jnp.float32),
        mesh=mesh,
        scratch_types=[
            pltpu.VMEM((1, 128), jnp.int32),
            pltpu.VMEM((32, D), jnp.float32),
            pltpu.VMEM((32, D), jnp.float32),
            pltpu.VMEM((16, D), jnp.float32),
            pltpu.VMEM((16, D), jnp.float32),
            pltpu.SemaphoreType.DMA,
            pltpu.SemaphoreType.DMA,
            pltpu.SemaphoreType.DMA,
            pltpu.SemaphoreType.DMA,
        ],
    )
    return f(cidx32, scaled)


@jax.jit
def kernel(x, Wg, W1, b1, W2, b2):
    dest2, gate2 = _run_router(x, Wg)
    dest32 = dest2.reshape(32, 128)
    gate32 = gate2.reshape(32, 128)
    tok32 = jnp.repeat(jnp.arange(T, dtype=jnp.int32), 2).reshape(32, 128)

    buf, gate_slot = _run_dispatch(dest32, tok32, gate32, x)

    b1r = b1.reshape(E, 1, DFF)
    b2r = b2.reshape(E, 1, D)
    scaled = _run_ffn(buf, W1, b1r, W2, b2r, gate_slot.reshape(S_PAD, 1))

    return _run_combine(dest32, scaled)
